# Initial kernel scaffold; baseline (speedup 1.0000x reference)
import jax
import jax.numpy as jnp
from jax.experimental import pallas as pl


def kernel(pred1_logits, pred2_logits, label_table):
    raise NotImplementedError("write your pallas kernel here")


# speedup vs baseline: 2.1867x; 2.1867x over previous
import functools

import jax
import jax.numpy as jnp
from jax.experimental import pallas as pl

_C1 = 10
_BB = 2048


def _loss_body(nblocks, batch, p1_ref, p2_ref, t_ref, out_ref):
    i = pl.program_id(0)
    p1 = p1_ref[...]
    p2 = p2_ref[...]
    tbl = t_ref[...]

    m = jnp.max(p2, axis=1, keepdims=True)
    lse = jnp.log(jnp.sum(jnp.exp(p2 - m), axis=1)) + m[:, 0]

    m1 = jnp.max(p1, axis=1, keepdims=True)
    ids = jax.lax.broadcasted_iota(jnp.int32, p1.shape, 1)
    cand = jnp.where(p1 == m1, ids, _C1)
    a = jnp.min(cand, axis=1)
    oh = (ids == a[:, None]).astype(jnp.float32)

    scores = jax.lax.dot_general(
        p2, tbl, (((1,), (1,)), ((), ())), preferred_element_type=jnp.float32
    )
    sel = jnp.sum(oh * scores, axis=1)

    tsum = jnp.sum(tbl, axis=1)
    mass = jnp.sum(oh * tsum[None, :], axis=1)

    part = jnp.sum(mass * lse - sel) * (1.0 / batch)

    @pl.when(i == 0)
    def _init():
        out_ref[...] = jnp.zeros_like(out_ref)

    out_ref[...] += jnp.reshape(part, (1, 1))


def kernel(pred1_logits, pred2_logits, label_table):
    batch, c1 = pred1_logits.shape
    _, c2 = pred2_logits.shape
    nblocks = batch // _BB

    out = pl.pallas_call(
        functools.partial(_loss_body, nblocks, batch),
        grid=(nblocks,),
        in_specs=[
            pl.BlockSpec((_BB, c1), lambda i: (i, 0)),
            pl.BlockSpec((_BB, c2), lambda i: (i, 0)),
            pl.BlockSpec((c1, c2), lambda i: (0, 0)),
        ],
        out_specs=pl.BlockSpec((1, 1), lambda i: (0, 0)),
        out_shape=jax.ShapeDtypeStruct((1, 1), jnp.float32),
    )(pred1_logits, pred2_logits, label_table)
    return out[0, 0]
